# Initial kernel scaffold; baseline (speedup 1.0000x reference)
import jax
import jax.numpy as jnp
from jax.experimental import pallas as pl


def kernel(x, edge_index, W_o, b_o, W_att, b_att):
    raise NotImplementedError("write your pallas kernel here")


# speedup vs baseline: 31.0287x; 31.0287x over previous
import functools

import jax
import jax.numpy as jnp
from jax import lax
from jax.experimental import pallas as pl
from jax.experimental.pallas import tpu as pltpu
from jax.experimental.pallas import tpu_sc as plsc

N = 10000
E = 320000
D = 128

NC = 2
NS = 16
NW = NC * NS
EW = E // NW
CH = 80
NCH = EW // CH

_F32 = jnp.float32


def _tc_prep_body(x_ref, wo_ref, bo_ref, watt_ref, batt_ref, ea_ref, y_ref):
    wv = jnp.dot(wo_ref[...], watt_ref[...], preferred_element_type=_F32)
    c0 = jnp.dot(bo_ref[...], watt_ref[...], preferred_element_type=_F32)
    a = jnp.dot(x_ref[...], wv, preferred_element_type=_F32) + c0 + batt_ref[0, 0]
    ea = jnp.exp(a - jnp.max(a))
    ea_ref[...] = ea
    y_ref[...] = x_ref[...] * ea


_tc_prep = pl.pallas_call(
    _tc_prep_body,
    out_shape=[
        jax.ShapeDtypeStruct((N, 1), _F32),
        jax.ShapeDtypeStruct((N, D), _F32),
    ],
)


def _sc_body(ea_hbm, src_hbm, dst_hbm, y_hbm, dp_hbm, hp_hbm,
             srcv, dstv, valsv, rowsv, zbuf1, dacc, hacc, sem, sem2):
    c = lax.axis_index("c")
    s = lax.axis_index("s")
    wid = c * NS + s

    pltpu.sync_copy(src_hbm.at[wid], srcv)
    pltpu.sync_copy(dst_hbm.at[wid], dstv)

    zeros16 = jnp.zeros((16,), _F32)

    def _zrow(j, _):
        for k in range(D // 16):
            rowsv[j, pl.ds(k * 16, 16)] = zeros16
        return 0

    lax.fori_loop(0, CH, _zrow, 0)

    def _z1(i, _):
        zbuf1[pl.ds(i * 16, 16)] = zeros16
        return 0

    lax.fori_loop(0, 40, _z1, 0)

    base = pl.multiple_of(s * 624, 8)
    pltpu.sync_copy(zbuf1, dacc.at[pl.ds(base, 640)])
    for k in range(8):
        pltpu.sync_copy(rowsv, hacc.at[pl.ds(base + k * CH, CH)])

    plsc.subcore_barrier()

    def _rrow(j, _):
        cps = pltpu.async_copy(ea_hbm.at[srcv.at[j]], valsv, sem2)
        cpr = pltpu.async_copy(y_hbm.at[srcv.at[j]], rowsv, sem)
        cps.wait()
        cpr.wait()
        pltpu.sync_copy(valsv, dacc.at[dstv.at[j]], add=True)
        pltpu.sync_copy(rowsv, hacc.at[dstv.at[j]], add=True)
        return 0

    lax.fori_loop(0, NCH, _rrow, 0)

    plsc.subcore_barrier()

    dpo = pl.multiple_of(c * N + s * 624, 8)
    pltpu.sync_copy(dacc.at[pl.ds(base, 640)], zbuf1)
    pltpu.sync_copy(zbuf1, dp_hbm.at[pl.ds(dpo, 640)])
    for k in range(8):
        pltpu.sync_copy(hacc.at[pl.ds(base + k * CH, CH)],
                        hp_hbm.at[c, pl.ds(base + k * CH, CH)])


_sc_edges = pl.kernel(
    _sc_body,
    out_type=[
        jax.ShapeDtypeStruct((NC * N,), _F32),
        jax.ShapeDtypeStruct((NC, N, D), _F32),
    ],
    mesh=plsc.VectorSubcoreMesh(core_axis_name="c", subcore_axis_name="s"),
    scratch_types=[
        pltpu.VMEM((NCH, CH), jnp.int32),
        pltpu.VMEM((NCH, CH), jnp.int32),
        pltpu.VMEM((CH,), _F32),
        pltpu.VMEM((CH, D), _F32),
        pltpu.VMEM((640,), _F32),
        pltpu.VMEM_SHARED((N,), _F32),
        pltpu.VMEM_SHARED((N, D), _F32),
        pltpu.SemaphoreType.DMA,
        pltpu.SemaphoreType.DMA,
    ],
)


def _tc_fin_body(hp_ref, dp_ref, o_ref):
    d = dp_ref[0] + dp_ref[1]
    o_ref[...] = (hp_ref[0] + hp_ref[1]) * (1.0 / (d + 1e-16))


_tc_fin = pl.pallas_call(
    _tc_fin_body,
    grid=(10,),
    in_specs=[
        pl.BlockSpec((NC, N // 10, D), lambda i: (0, i, 0)),
        pl.BlockSpec((NC, N // 10, 1), lambda i: (0, i, 0)),
    ],
    out_specs=pl.BlockSpec((N // 10, D), lambda i: (i, 0)),
    out_shape=jax.ShapeDtypeStruct((N, D), _F32),
)


@jax.jit
def kernel(x, edge_index, W_o, b_o, W_att, b_att):
    src_g = edge_index[0].reshape(NW, NCH, CH)
    dst_g = edge_index[1].reshape(NW, NCH, CH)
    ea, y = _tc_prep(x, W_o, b_o.reshape(1, D), W_att, b_att.reshape(1, 1))
    dp, hp = _sc_edges(ea.reshape(N), src_g, dst_g, y)
    return _tc_fin(hp, dp.reshape(NC, N, 1))


# speedup vs baseline: 40.3796x; 1.3014x over previous
import functools

import jax
import jax.numpy as jnp
from jax import lax
from jax.experimental import pallas as pl
from jax.experimental.pallas import tpu as pltpu
from jax.experimental.pallas import tpu_sc as plsc

N = 10000
E = 320000
D = 128

NC = 2
NS = 16
NW = NC * NS
EW = E // NW
CH = 100
NCH = EW // CH
BC = 20
NB = NCH // BC

_F32 = jnp.float32


def _tc_prep_body(x_ref, wo_ref, bo_ref, watt_ref, batt_ref, ea_ref, y_ref):
    wv = jnp.dot(wo_ref[...], watt_ref[...], preferred_element_type=_F32)
    c0 = jnp.dot(bo_ref[...], watt_ref[...], preferred_element_type=_F32)
    a = jnp.dot(x_ref[...], wv, preferred_element_type=_F32) + c0 + batt_ref[0, 0]
    ea = jnp.exp(a - jnp.max(a))
    ea_ref[...] = ea
    y_ref[...] = x_ref[...] * ea


_tc_prep = pl.pallas_call(
    _tc_prep_body,
    out_shape=[
        jax.ShapeDtypeStruct((N, 1), _F32),
        jax.ShapeDtypeStruct((N, D), _F32),
    ],
)


def _sc_body(ea_hbm, src_hbm, dst_hbm, y_hbm, dp_hbm, hp_hbm,
             srcv, dstv, vals0, vals1, rows0, rows1, zbuf1, dacc, hacc,
             semr, sems, semss):
    c = lax.axis_index("c")
    s = lax.axis_index("s")
    wid = c * NS + s

    zeros16 = jnp.zeros((16,), _F32)

    def _zrow(j, _):
        for k in range(D // 16):
            rows0[j, pl.ds(k * 16, 16)] = zeros16
        return 0

    lax.fori_loop(0, CH, _zrow, 0)

    def _z1(i, _):
        zbuf1[pl.ds(i * 16, 16)] = zeros16
        return 0

    lax.fori_loop(0, 40, _z1, 0)

    base = pl.multiple_of(s * 624, 8)
    pltpu.sync_copy(zbuf1, dacc.at[pl.ds(base, 640)])
    for k in range(8):
        pltpu.sync_copy(rows0.at[pl.ds(0, 80)], hacc.at[pl.ds(base + k * 80, 80)])

    plsc.subcore_barrier()

    bufs = ((rows0, vals0), (rows1, vals1))

    def _issue_g(j, b):
        pltpu.async_copy(y_hbm.at[srcv.at[j]], bufs[b][0], semr)
        pltpu.async_copy(ea_hbm.at[srcv.at[j]], bufs[b][1], sems)

    def _wait_g(b):
        pltpu.make_async_copy(y_hbm.at[srcv.at[0]], bufs[b][0], semr).wait()
        pltpu.make_async_copy(ea_hbm.at[srcv.at[0]], bufs[b][1], sems).wait()

    def _drain_ss(b):
        pltpu.make_async_copy(ea_hbm.at[srcv.at[0]], bufs[b][1], semss).wait()

    def _step(j, b, drain=True, issue=True):
        _wait_g(b)
        if drain:
            _drain_ss(1 - b)
        if issue:
            _issue_g(j + 1, 1 - b)
        pltpu.async_copy(bufs[b][1], dacc.at[dstv.at[j]], semss, add=True)
        pltpu.sync_copy(bufs[b][0], hacc.at[dstv.at[j]], add=True)

    for blk in range(NB):
        pltpu.sync_copy(src_hbm.at[wid, blk], srcv)
        pltpu.sync_copy(dst_hbm.at[wid, blk], dstv)
        _issue_g(0, 0)
        _step(0, 0, drain=(blk > 0))

        def _loop(i, _):
            j = 1 + 2 * i
            _step(j, 1)
            _step(j + 1, 0)
            return 0

        lax.fori_loop(0, (BC - 4) // 2, _loop, 0)
        _step(BC - 3, 1)
        _step(BC - 2, 0)
        _step(BC - 1, 1, issue=False)
    _drain_ss(1)

    plsc.subcore_barrier()

    dpo = pl.multiple_of(c * N + s * 624, 8)
    pltpu.sync_copy(dacc.at[pl.ds(base, 640)], zbuf1)
    pltpu.sync_copy(zbuf1, dp_hbm.at[pl.ds(dpo, 640)])
    for k in range(8):
        pltpu.sync_copy(hacc.at[pl.ds(base + k * 80, 80)],
                        hp_hbm.at[c, pl.ds(base + k * 80, 80)])


_sc_edges = pl.kernel(
    _sc_body,
    out_type=[
        jax.ShapeDtypeStruct((NC * N,), _F32),
        jax.ShapeDtypeStruct((NC, N, D), _F32),
    ],
    mesh=plsc.VectorSubcoreMesh(core_axis_name="c", subcore_axis_name="s"),
    scratch_types=[
        pltpu.VMEM((BC, CH), jnp.int32),
        pltpu.VMEM((BC, CH), jnp.int32),
        pltpu.VMEM((CH,), _F32),
        pltpu.VMEM((CH,), _F32),
        pltpu.VMEM((CH, D), _F32),
        pltpu.VMEM((CH, D), _F32),
        pltpu.VMEM((640,), _F32),
        pltpu.VMEM_SHARED((N,), _F32),
        pltpu.VMEM_SHARED((N, D), _F32),
        pltpu.SemaphoreType.DMA,
        pltpu.SemaphoreType.DMA,
        pltpu.SemaphoreType.DMA,
    ],
)


def _tc_fin_body(hp_ref, dp_ref, o_ref):
    d = dp_ref[0] + dp_ref[1]
    o_ref[...] = (hp_ref[0] + hp_ref[1]) * (1.0 / (d + 1e-16))


_tc_fin = pl.pallas_call(
    _tc_fin_body,
    grid=(10,),
    in_specs=[
        pl.BlockSpec((NC, N // 10, D), lambda i: (0, i, 0)),
        pl.BlockSpec((NC, N // 10, 1), lambda i: (0, i, 0)),
    ],
    out_specs=pl.BlockSpec((N // 10, D), lambda i: (i, 0)),
    out_shape=jax.ShapeDtypeStruct((N, D), _F32),
)


@jax.jit
def kernel(x, edge_index, W_o, b_o, W_att, b_att):
    src_g = edge_index[0].reshape(NW, NB, BC, CH)
    dst_g = edge_index[1].reshape(NW, NB, BC, CH)
    ea, y = _tc_prep(x, W_o, b_o.reshape(1, D), W_att, b_att.reshape(1, 1))
    dp, hp = _sc_edges(ea.reshape(N), src_g, dst_g, y)
    return _tc_fin(hp, dp.reshape(NC, N, 1))


# speedup vs baseline: 43.0416x; 1.0659x over previous
import functools

import jax
import jax.numpy as jnp
from jax import lax
from jax.experimental import pallas as pl
from jax.experimental.pallas import tpu as pltpu
from jax.experimental.pallas import tpu_sc as plsc

N = 10000
E = 320000
D = 128

NC = 2
NS = 16
NW = NC * NS
EW = E // NW
CH = 125
NCH = EW // CH
BC = 20
NB = NCH // BC

_F32 = jnp.float32


def _tc_prep_body(x_ref, wo_ref, bo_ref, watt_ref, batt_ref, ea_ref, y_ref):
    wv = jnp.dot(wo_ref[...], watt_ref[...], preferred_element_type=_F32)
    c0 = jnp.dot(bo_ref[...], watt_ref[...], preferred_element_type=_F32)
    a = jnp.dot(x_ref[...], wv, preferred_element_type=_F32) + c0 + batt_ref[0, 0]
    ea = jnp.exp(a - jnp.max(a))
    ea_ref[...] = ea
    y_ref[...] = x_ref[...] * ea


_tc_prep = pl.pallas_call(
    _tc_prep_body,
    out_shape=[
        jax.ShapeDtypeStruct((N, 1), _F32),
        jax.ShapeDtypeStruct((N, D), _F32),
    ],
)


def _sc_body(ea_hbm, src_hbm, dst_hbm, y_hbm, dp_hbm, hp_hbm,
             srcv, dstv, vals0, vals1, rows0, rows1, zbuf1, dacc, hacc,
             semr, sems, semss):
    c = lax.axis_index("c")
    s = lax.axis_index("s")
    wid = c * NS + s

    zeros16 = jnp.zeros((16,), _F32)

    def _zrow(j, _):
        for k in range(D // 16):
            rows0[j, pl.ds(k * 16, 16)] = zeros16
        return 0

    lax.fori_loop(0, CH, _zrow, 0)

    def _z1(i, _):
        zbuf1[pl.ds(i * 16, 16)] = zeros16
        return 0

    lax.fori_loop(0, 40, _z1, 0)

    base = pl.multiple_of(s * 624, 8)
    pltpu.sync_copy(zbuf1, dacc.at[pl.ds(base, 640)])
    for k in range(8):
        pltpu.sync_copy(rows0.at[pl.ds(0, 80)], hacc.at[pl.ds(base + k * 80, 80)])

    plsc.subcore_barrier()

    bufs = ((rows0, vals0), (rows1, vals1))

    def _issue_g(j, b):
        pltpu.async_copy(y_hbm.at[srcv.at[j]], bufs[b][0], semr)
        pltpu.async_copy(ea_hbm.at[srcv.at[j]], bufs[b][1], sems)

    def _wait_g(b):
        pltpu.make_async_copy(y_hbm.at[srcv.at[0]], bufs[b][0], semr).wait()
        pltpu.make_async_copy(ea_hbm.at[srcv.at[0]], bufs[b][1], sems).wait()

    def _drain_ss(b):
        pltpu.make_async_copy(ea_hbm.at[srcv.at[0]], bufs[b][1], semss).wait()

    def _step(j, b, drain=True, issue=True):
        _wait_g(b)
        if drain:
            _drain_ss(1 - b)
        if issue:
            _issue_g(j + 1, 1 - b)
        pltpu.async_copy(bufs[b][1], dacc.at[dstv.at[j]], semss, add=True)
        pltpu.sync_copy(bufs[b][0], hacc.at[dstv.at[j]], add=True)

    for blk in range(NB):
        pltpu.sync_copy(src_hbm.at[wid, blk], srcv)
        pltpu.sync_copy(dst_hbm.at[wid, blk], dstv)
        _issue_g(0, 0)
        _step(0, 0, drain=(blk > 0))

        def _loop(i, _):
            j = 1 + 2 * i
            _step(j, 1)
            _step(j + 1, 0)
            return 0

        lax.fori_loop(0, (BC - 4) // 2, _loop, 0)
        _step(BC - 3, 1)
        _step(BC - 2, 0)
        _step(BC - 1, 1, issue=False)
    _drain_ss(1)

    plsc.subcore_barrier()

    dpo = pl.multiple_of(c * N + s * 624, 8)
    pltpu.sync_copy(dacc.at[pl.ds(base, 640)], zbuf1)
    pltpu.sync_copy(zbuf1, dp_hbm.at[pl.ds(dpo, 640)])
    for k in range(8):
        pltpu.sync_copy(hacc.at[pl.ds(base + k * 80, 80)],
                        hp_hbm.at[c, pl.ds(base + k * 80, 80)])


_sc_edges = pl.kernel(
    _sc_body,
    out_type=[
        jax.ShapeDtypeStruct((NC * N,), _F32),
        jax.ShapeDtypeStruct((NC, N, D), _F32),
    ],
    mesh=plsc.VectorSubcoreMesh(core_axis_name="c", subcore_axis_name="s"),
    scratch_types=[
        pltpu.VMEM((BC, CH), jnp.int32),
        pltpu.VMEM((BC, CH), jnp.int32),
        pltpu.VMEM((CH,), _F32),
        pltpu.VMEM((CH,), _F32),
        pltpu.VMEM((CH, D), _F32),
        pltpu.VMEM((CH, D), _F32),
        pltpu.VMEM((640,), _F32),
        pltpu.VMEM_SHARED((N,), _F32),
        pltpu.VMEM_SHARED((N, D), _F32),
        pltpu.SemaphoreType.DMA,
        pltpu.SemaphoreType.DMA,
        pltpu.SemaphoreType.DMA,
    ],
)


def _tc_fin_body(hp_ref, dp_ref, o_ref):
    d = dp_ref[0] + dp_ref[1]
    o_ref[...] = (hp_ref[0] + hp_ref[1]) * (1.0 / (d + 1e-16))


_tc_fin = pl.pallas_call(
    _tc_fin_body,
    grid=(10,),
    in_specs=[
        pl.BlockSpec((NC, N // 10, D), lambda i: (0, i, 0)),
        pl.BlockSpec((NC, N // 10, 1), lambda i: (0, i, 0)),
    ],
    out_specs=pl.BlockSpec((N // 10, D), lambda i: (i, 0)),
    out_shape=jax.ShapeDtypeStruct((N, D), _F32),
)


@jax.jit
def kernel(x, edge_index, W_o, b_o, W_att, b_att):
    src_g = edge_index[0].reshape(NW, NB, BC, CH)
    dst_g = edge_index[1].reshape(NW, NB, BC, CH)
    ea, y = _tc_prep(x, W_o, b_o.reshape(1, D), W_att, b_att.reshape(1, 1))
    dp, hp = _sc_edges(ea.reshape(N), src_g, dst_g, y)
    return _tc_fin(hp, dp.reshape(NC, N, 1))
